# Initial kernel scaffold; baseline (speedup 1.0000x reference)
import jax
import jax.numpy as jnp
from jax.experimental import pallas as pl


def kernel(x, edge_index, W1, b1, K1, kb1, W2, b2, K2, kb2, W3, b3, K3, kb3, Wfc, bfc):
    raise NotImplementedError("write your pallas kernel here")


# speedup vs baseline: 16.6678x; 16.6678x over previous
import functools

import jax
import jax.numpy as jnp
from jax import lax
from jax.experimental import pallas as pl
from jax.experimental.pallas import tpu as pltpu
from jax.experimental.pallas import tpu_sc as plsc

_N = 25000
_NTILES = 16
_TROWS = 1664
_NACC = _NTILES * _TROWS
_E = 400000
_CW = 128
_NW = 32
_CHUNKS = 98
_EPAD = _NW * _CHUNKS * _CW
_R = 1000
_GRID = _N // _R

_SC_PARAMS = pltpu.CompilerParams(use_tc_tiling_on_sc=False)
_MESH = dict(core_axis_name="c", subcore_axis_name="s")


def _make_agg1(D):
    @functools.partial(
        pl.kernel,
        out_type=jax.ShapeDtypeStruct((2, _NACC, D), jnp.float32),
        mesh=plsc.VectorSubcoreMesh(**_MESH),
        compiler_params=_SC_PARAMS,
        scratch_types=[
            pltpu.VMEM((_CHUNKS, _CW), jnp.int32),
            pltpu.VMEM((_CHUNKS, _CW), jnp.int32),
            pltpu.VMEM((_CW, D), jnp.float32),
            pltpu.VMEM_SHARED((_NACC, D), jnp.float32),
            pltpu.SemaphoreType.DMA,
        ],
    )
    def agg(xs_hbm, src_hbm, dst_hbm, zero_hbm, out_hbm, srcb, dstb, gbuf,
            acc, sem):
        cid = lax.axis_index("c")
        sid = lax.axis_index("s")
        wid = cid * _NTILES + sid
        lo = sid * _TROWS
        pltpu.sync_copy(src_hbm.at[wid], srcb)
        pltpu.sync_copy(dst_hbm.at[wid], dstb)
        pltpu.sync_copy(zero_hbm, acc.at[pl.ds(lo, _TROWS)])
        plsc.subcore_barrier()

        def body(c, carry):
            pltpu.async_copy(xs_hbm.at[srcb.at[c]], gbuf, sem).wait()
            pltpu.sync_copy(gbuf, acc.at[dstb.at[c]], add=True)
            return carry

        lax.fori_loop(0, _CHUNKS, body, 0)
        plsc.subcore_barrier()
        pltpu.sync_copy(acc.at[pl.ds(lo, _TROWS)],
                        out_hbm.at[cid, pl.ds(lo, _TROWS)])

    return agg


def _make_agg2(D):
    @functools.partial(
        pl.kernel,
        out_type=[jax.ShapeDtypeStruct((2, _NACC, D), jnp.float32)] * 2,
        mesh=plsc.VectorSubcoreMesh(**_MESH),
        compiler_params=_SC_PARAMS,
        scratch_types=[
            pltpu.VMEM((_CHUNKS, _CW), jnp.int32),
            pltpu.VMEM((_CHUNKS, _CW), jnp.int32),
            pltpu.VMEM((_CW, D), jnp.float32),
            pltpu.VMEM((_CW, D), jnp.float32),
            pltpu.VMEM_SHARED((_NACC, D), jnp.float32),
            pltpu.VMEM_SHARED((_NACC, D), jnp.float32),
            pltpu.SemaphoreType.DMA,
            pltpu.SemaphoreType.DMA,
        ],
    )
    def agg(xa_hbm, xb_hbm, src_hbm, dst_hbm, zero_hbm, outa_hbm, outb_hbm,
            srcb, dstb, gba, gbb, acca, accb, sema, semb):
        cid = lax.axis_index("c")
        sid = lax.axis_index("s")
        wid = cid * _NTILES + sid
        lo = sid * _TROWS
        pltpu.sync_copy(src_hbm.at[wid], srcb)
        pltpu.sync_copy(dst_hbm.at[wid], dstb)
        pltpu.sync_copy(zero_hbm, acca.at[pl.ds(lo, _TROWS)])
        pltpu.sync_copy(zero_hbm, accb.at[pl.ds(lo, _TROWS)])
        plsc.subcore_barrier()

        def body(c, carry):
            ca = pltpu.async_copy(xa_hbm.at[srcb.at[c]], gba, sema)
            cb = pltpu.async_copy(xb_hbm.at[srcb.at[c]], gbb, semb)
            ca.wait()
            pltpu.sync_copy(gba, acca.at[dstb.at[c]], add=True)
            cb.wait()
            pltpu.sync_copy(gbb, accb.at[dstb.at[c]], add=True)
            return carry

        lax.fori_loop(0, _CHUNKS, body, 0)
        plsc.subcore_barrier()
        pltpu.sync_copy(acca.at[pl.ds(lo, _TROWS)],
                        outa_hbm.at[cid, pl.ds(lo, _TROWS)])
        pltpu.sync_copy(accb.at[pl.ds(lo, _TROWS)],
                        outb_hbm.at[cid, pl.ds(lo, _TROWS)])

    return agg


def _make_deg():
    @functools.partial(
        pl.kernel,
        out_type=jax.ShapeDtypeStruct((2, _NACC, 16), jnp.float32),
        mesh=plsc.VectorSubcoreMesh(**_MESH),
        compiler_params=_SC_PARAMS,
        scratch_types=[
            pltpu.VMEM((_CHUNKS, _CW), jnp.int32),
            pltpu.VMEM((_CW, 16), jnp.float32),
            pltpu.VMEM_SHARED((_NACC, 16), jnp.float32),
        ],
    )
    def deg(ones_hbm, dst_hbm, zero_hbm, out_hbm, dstb, obuf, acc):
        cid = lax.axis_index("c")
        sid = lax.axis_index("s")
        wid = cid * _NTILES + sid
        lo = sid * _TROWS
        pltpu.sync_copy(dst_hbm.at[wid], dstb)
        pltpu.sync_copy(ones_hbm, obuf)
        pltpu.sync_copy(zero_hbm, acc.at[pl.ds(lo, _TROWS)])
        plsc.subcore_barrier()

        def body(c, carry):
            pltpu.sync_copy(obuf, acc.at[dstb.at[c]], add=True)
            return carry

        lax.fori_loop(0, _CHUNKS, body, 0)
        plsc.subcore_barrier()
        pltpu.sync_copy(acc.at[pl.ds(lo, _TROWS)],
                        out_hbm.at[cid, pl.ds(lo, _TROWS)])

    return deg


def _dis_body(deg_ref, x_ref, dis_ref, xs_ref):
    d = deg_ref[0, :, 0:1] + deg_ref[1, :, 0:1] + 1.0
    dis = lax.rsqrt(d)
    dis_ref[...] = dis
    xs_ref[...] = dis * x_ref[...]


def _dis_kernel(deg_out, xf):
    return pl.pallas_call(
        _dis_body,
        grid=(_GRID,),
        in_specs=[
            pl.BlockSpec((2, _R, 16), lambda i: (0, i, 0)),
            pl.BlockSpec((_R, 16), lambda i: (i, 0)),
        ],
        out_specs=[
            pl.BlockSpec((_R, 1), lambda i: (i, 0)),
            pl.BlockSpec((_R, 16), lambda i: (i, 0)),
        ],
        out_shape=[
            jax.ShapeDtypeStruct((_N, 1), jnp.float32),
            jax.ShapeDtypeStruct((_N, 16), jnp.float32),
        ],
    )(deg_out, xf)


def _make_block(Din, Dout, n_in, n_out):
    P = Din // n_in

    def body(*refs):
        dis_ref = refs[0]
        xs_refs = refs[1:1 + n_in]
        ag_refs = refs[1 + n_in:1 + 2 * n_in]
        w_ref, b_ref, k0, k1, k2, kb_ref = refs[1 + 2 * n_in:7 + 2 * n_in]
        outs = refs[7 + 2 * n_in:]
        if n_in == 1:
            xs = xs_refs[0][...]
            a = ag_refs[0][0] + ag_refs[0][1]
        else:
            xs = jnp.concatenate([r[...] for r in xs_refs], axis=-1)
            a = jnp.concatenate([r[0] + r[1] for r in ag_refs], axis=-1)
        dis = dis_ref[...]
        m = dis * (a + xs)
        g = jnp.dot(m, w_ref[...], preferred_element_type=jnp.float32)
        g = g + b_ref[...]
        t = lax.broadcasted_iota(jnp.int32, (_R, 1), 0) % 10
        zrow = jnp.zeros((1, Dout), jnp.float32)
        gm1 = jnp.where(t == 0, 0.0,
                        jnp.concatenate([zrow, g[:-1]], axis=0))
        gp1 = jnp.where(t == 9, 0.0,
                        jnp.concatenate([g[1:], zrow], axis=0))
        y = (jnp.dot(g, k1[...], preferred_element_type=jnp.float32)
             + jnp.dot(gm1, k0[...], preferred_element_type=jnp.float32)
             + jnp.dot(gp1, k2[...], preferred_element_type=jnp.float32)
             + kb_ref[...])
        y = jnp.maximum(y, 0.0)
        if n_out == 0:
            ps = jnp.sum(y.reshape(_R // 8, 8, Dout), axis=0)
            i = pl.program_id(0)

            @pl.when(i == 0)
            def _():
                outs[0][...] = jnp.zeros_like(outs[0])

            outs[0][...] += ps
        else:
            ys = dis * y
            q = Dout // n_out
            for j in range(n_out):
                outs[j][...] = ys[:, j * q:(j + 1) * q]

    full = lambda i: (0, 0)
    in_specs = [pl.BlockSpec((_R, 1), lambda i: (i, 0))]
    in_specs += [pl.BlockSpec((_R, P), lambda i: (i, 0))] * n_in
    in_specs += [pl.BlockSpec((2, _R, P), lambda i: (0, i, 0))] * n_in
    in_specs += [
        pl.BlockSpec((Din, Dout), full),
        pl.BlockSpec((1, Dout), full),
        pl.BlockSpec((Dout, Dout), full),
        pl.BlockSpec((Dout, Dout), full),
        pl.BlockSpec((Dout, Dout), full),
        pl.BlockSpec((1, Dout), full),
    ]
    if n_out == 0:
        out_specs = [pl.BlockSpec((8, Dout), full)]
        out_shape = [jax.ShapeDtypeStruct((8, Dout), jnp.float32)]
    else:
        q = Dout // n_out
        out_specs = [pl.BlockSpec((_R, q), lambda i: (i, 0))] * n_out
        out_shape = [jax.ShapeDtypeStruct((_N, q), jnp.float32)] * n_out

    def run(*args):
        return pl.pallas_call(
            body,
            grid=(_GRID,),
            in_specs=in_specs,
            out_specs=out_specs,
            out_shape=out_shape,
        )(*args)

    return run


_deg_k = _make_deg()
_agg16 = _make_agg1(16)
_agg32 = _make_agg1(32)
_blk1 = _make_block(16, 64, 1, 2)
_blk2 = _make_block(64, 128, 2, 4)
_blk3 = _make_block(128, 256, 4, 0)


def kernel(x, edge_index, W1, b1, K1, kb1, W2, b2, K2, kb2, W3, b3, K3, kb3,
           Wfc, bfc):
    xf = x.reshape(_N, 16)
    src = edge_index[0].astype(jnp.int32)
    dst = edge_index[1].astype(jnp.int32)
    pad = _EPAD - _E
    padi = jnp.arange(pad, dtype=jnp.int32)
    src_p = jnp.concatenate([src, padi % _N]).reshape(_NW, _CHUNKS, _CW)
    dst_p = jnp.concatenate(
        [dst, _N + 600 + (padi % 1024)]).reshape(_NW, _CHUNKS, _CW)

    ones16 = jnp.ones((_CW, 16), jnp.float32)
    z16 = jnp.zeros((_TROWS, 16), jnp.float32)
    z32 = jnp.zeros((_TROWS, 32), jnp.float32)

    deg_out = _deg_k(ones16, dst_p, z16)
    dis, xs1 = _dis_kernel(deg_out, xf)

    kt = lambda K, h: K[:, :, h, 0].T
    rs = lambda v: v.reshape(1, -1)
    agg1 = _agg16(xs1, src_p, dst_p, z16)
    y2a, y2b = _blk1(dis, xs1, agg1, W1, rs(b1), kt(K1, 0), kt(K1, 1),
                     kt(K1, 2), rs(kb1))
    ag2a = _agg32(y2a, src_p, dst_p, z32)
    ag2b = _agg32(y2b, src_p, dst_p, z32)
    y3a, y3b, y3c, y3d = _blk2(dis, y2a, y2b, ag2a, ag2b, W2, rs(b2),
                               kt(K2, 0), kt(K2, 1), kt(K2, 2), rs(kb2))
    ag3a = _agg32(y3a, src_p, dst_p, z32)
    ag3b = _agg32(y3b, src_p, dst_p, z32)
    ag3c = _agg32(y3c, src_p, dst_p, z32)
    ag3d = _agg32(y3d, src_p, dst_p, z32)
    cs = _blk3(dis, y3a, y3b, y3c, y3d, ag3a, ag3b, ag3c, ag3d, W3, rs(b3),
               kt(K3, 0), kt(K3, 1), kt(K3, 2), rs(kb3))[0]
    colmean = jnp.sum(cs, axis=0) * (1.0 / _N)
    return (colmean @ Wfc + bfc)[None, :]


# speedup vs baseline: 26.6501x; 1.5989x over previous
import functools

import jax
import jax.numpy as jnp
from jax import lax
from jax.experimental import pallas as pl
from jax.experimental.pallas import tpu as pltpu
from jax.experimental.pallas import tpu_sc as plsc

_N = 25000
_NTILES = 16
_TROWS = 1664
_NACC = _NTILES * _TROWS
_E = 400000
_CW = 128
_NW = 32
_NBUF = 4
_CHUNKS = 100
_EPAD = _NW * _CHUNKS * _CW
_R = 1000
_GRID = _N // _R

_SC_PARAMS = pltpu.CompilerParams(use_tc_tiling_on_sc=False)
_MESH = dict(core_axis_name="c", subcore_axis_name="s")


def _make_agg1(D):
    @functools.partial(
        pl.kernel,
        out_type=jax.ShapeDtypeStruct((2, _NACC, D), jnp.float32),
        mesh=plsc.VectorSubcoreMesh(**_MESH),
        compiler_params=_SC_PARAMS,
        scratch_types=[
            pltpu.VMEM((_CHUNKS, _CW), jnp.int32),
            pltpu.VMEM((_CHUNKS, _CW), jnp.int32),
            [pltpu.VMEM((_CW, D), jnp.float32)] * _NBUF,
            [pltpu.SemaphoreType.DMA] * _NBUF,
            [pltpu.SemaphoreType.DMA] * _NBUF,
            pltpu.VMEM_SHARED((_NACC, D), jnp.float32),
        ],
    )
    def agg(xs_hbm, src_hbm, dst_hbm, zero_hbm, out_hbm, srcb, dstb, gbufs,
            gsems, ssems, acc):
        cid = lax.axis_index("c")
        sid = lax.axis_index("s")
        wid = cid * _NTILES + sid
        lo = sid * _TROWS
        pltpu.sync_copy(src_hbm.at[wid], srcb)
        pltpu.sync_copy(dst_hbm.at[wid], dstb)
        pltpu.sync_copy(zero_hbm, acc.at[pl.ds(lo, _TROWS)])
        plsc.subcore_barrier()

        def gath(c, b):
            pltpu.async_copy(xs_hbm.at[srcb.at[c]], gbufs[b], gsems[b])

        def gath_wait(c, b):
            pltpu.make_async_copy(xs_hbm.at[srcb.at[c]], gbufs[b],
                                  gsems[b]).wait()

        def scat(c, b):
            pltpu.async_copy(gbufs[b], acc.at[dstb.at[c]], ssems[b],
                             add=True)

        def scat_wait(c, b):
            pltpu.make_async_copy(gbufs[b], acc.at[dstb.at[c]],
                                  ssems[b]).wait()

        for b in range(_NBUF):
            gath(b, b)

        def body(i, carry):
            c0 = i * _NBUF
            for b in range(_NBUF):
                gath_wait(c0 + b, b)
                scat(c0 + b, b)
            for b in range(_NBUF):
                scat_wait(c0 + b, b)
                gath(c0 + _NBUF + b, b)
            return carry

        lax.fori_loop(0, _CHUNKS // _NBUF - 1, body, 0)
        c0 = _CHUNKS - _NBUF
        for b in range(_NBUF):
            gath_wait(c0 + b, b)
            scat(c0 + b, b)
        for b in range(_NBUF):
            scat_wait(c0 + b, b)
        plsc.subcore_barrier()
        pltpu.sync_copy(acc.at[pl.ds(lo, _TROWS)],
                        out_hbm.at[cid, pl.ds(lo, _TROWS)])

    return agg


def _make_deg():
    @functools.partial(
        pl.kernel,
        out_type=jax.ShapeDtypeStruct((2, _NACC, 16), jnp.float32),
        mesh=plsc.VectorSubcoreMesh(**_MESH),
        compiler_params=_SC_PARAMS,
        scratch_types=[
            pltpu.VMEM((_CHUNKS, _CW), jnp.int32),
            pltpu.VMEM((_CW, 16), jnp.float32),
            pltpu.VMEM_SHARED((_NACC, 16), jnp.float32),
        ],
    )
    def deg(ones_hbm, dst_hbm, zero_hbm, out_hbm, dstb, obuf, acc):
        cid = lax.axis_index("c")
        sid = lax.axis_index("s")
        wid = cid * _NTILES + sid
        lo = sid * _TROWS
        pltpu.sync_copy(dst_hbm.at[wid], dstb)
        pltpu.sync_copy(ones_hbm, obuf)
        pltpu.sync_copy(zero_hbm, acc.at[pl.ds(lo, _TROWS)])
        plsc.subcore_barrier()

        def body(c, carry):
            pltpu.sync_copy(obuf, acc.at[dstb.at[c]], add=True)
            return carry

        lax.fori_loop(0, _CHUNKS, body, 0)
        plsc.subcore_barrier()
        pltpu.sync_copy(acc.at[pl.ds(lo, _TROWS)],
                        out_hbm.at[cid, pl.ds(lo, _TROWS)])

    return deg


def _dis_body(deg_ref, x_ref, dis_ref, xs_ref):
    d = deg_ref[0, :, 0:1] + deg_ref[1, :, 0:1] + 1.0
    dis = lax.rsqrt(d)
    dis_ref[...] = dis
    xs_ref[...] = dis * x_ref[...]


def _dis_kernel(deg_out, xf):
    return pl.pallas_call(
        _dis_body,
        grid=(_GRID,),
        in_specs=[
            pl.BlockSpec((2, _R, 16), lambda i: (0, i, 0)),
            pl.BlockSpec((_R, 16), lambda i: (i, 0)),
        ],
        out_specs=[
            pl.BlockSpec((_R, 1), lambda i: (i, 0)),
            pl.BlockSpec((_R, 16), lambda i: (i, 0)),
        ],
        out_shape=[
            jax.ShapeDtypeStruct((_N, 1), jnp.float32),
            jax.ShapeDtypeStruct((_N, 16), jnp.float32),
        ],
    )(deg_out, xf)


def _make_block(Din, Dout, n_in, n_out):
    P = Din // n_in

    def body(*refs):
        dis_ref = refs[0]
        xs_refs = refs[1:1 + n_in]
        ag_refs = refs[1 + n_in:1 + 2 * n_in]
        w_ref, b_ref, k0, k1, k2, kb_ref = refs[1 + 2 * n_in:7 + 2 * n_in]
        outs = refs[7 + 2 * n_in:]
        if n_in == 1:
            xs = xs_refs[0][...]
            a = ag_refs[0][0] + ag_refs[0][1]
        else:
            xs = jnp.concatenate([r[...] for r in xs_refs], axis=-1)
            a = jnp.concatenate([r[0] + r[1] for r in ag_refs], axis=-1)
        dis = dis_ref[...]
        m = dis * (a + xs)
        g = jnp.dot(m, w_ref[...], preferred_element_type=jnp.float32)
        g = g + b_ref[...]
        t = lax.broadcasted_iota(jnp.int32, (_R, 1), 0) % 10
        zrow = jnp.zeros((1, Dout), jnp.float32)
        gm1 = jnp.where(t == 0, 0.0,
                        jnp.concatenate([zrow, g[:-1]], axis=0))
        gp1 = jnp.where(t == 9, 0.0,
                        jnp.concatenate([g[1:], zrow], axis=0))
        y = (jnp.dot(g, k1[...], preferred_element_type=jnp.float32)
             + jnp.dot(gm1, k0[...], preferred_element_type=jnp.float32)
             + jnp.dot(gp1, k2[...], preferred_element_type=jnp.float32)
             + kb_ref[...])
        y = jnp.maximum(y, 0.0)
        if n_out == 0:
            ps = jnp.sum(y.reshape(_R // 8, 8, Dout), axis=0)
            i = pl.program_id(0)

            @pl.when(i == 0)
            def _():
                outs[0][...] = jnp.zeros_like(outs[0])

            outs[0][...] += ps
        else:
            ys = dis * y
            q = Dout // n_out
            for j in range(n_out):
                outs[j][...] = ys[:, j * q:(j + 1) * q]

    full = lambda i: (0, 0)
    in_specs = [pl.BlockSpec((_R, 1), lambda i: (i, 0))]
    in_specs += [pl.BlockSpec((_R, P), lambda i: (i, 0))] * n_in
    in_specs += [pl.BlockSpec((2, _R, P), lambda i: (0, i, 0))] * n_in
    in_specs += [
        pl.BlockSpec((Din, Dout), full),
        pl.BlockSpec((1, Dout), full),
        pl.BlockSpec((Dout, Dout), full),
        pl.BlockSpec((Dout, Dout), full),
        pl.BlockSpec((Dout, Dout), full),
        pl.BlockSpec((1, Dout), full),
    ]
    if n_out == 0:
        out_specs = [pl.BlockSpec((8, Dout), full)]
        out_shape = [jax.ShapeDtypeStruct((8, Dout), jnp.float32)]
    else:
        q = Dout // n_out
        out_specs = [pl.BlockSpec((_R, q), lambda i: (i, 0))] * n_out
        out_shape = [jax.ShapeDtypeStruct((_N, q), jnp.float32)] * n_out

    def run(*args):
        return pl.pallas_call(
            body,
            grid=(_GRID,),
            in_specs=in_specs,
            out_specs=out_specs,
            out_shape=out_shape,
        )(*args)

    return run


_deg_k = _make_deg()
_agg16 = _make_agg1(16)
_agg32 = _make_agg1(32)
_blk1 = _make_block(16, 64, 1, 2)
_blk2 = _make_block(64, 128, 2, 4)
_blk3 = _make_block(128, 256, 4, 0)


def kernel(x, edge_index, W1, b1, K1, kb1, W2, b2, K2, kb2, W3, b3, K3, kb3,
           Wfc, bfc):
    xf = x.reshape(_N, 16)
    src = edge_index[0].astype(jnp.int32)
    dst = edge_index[1].astype(jnp.int32)
    pad = _EPAD - _E
    padi = jnp.arange(pad, dtype=jnp.int32)
    src_p = jnp.concatenate([src, padi % _N]).reshape(_NW, _CHUNKS, _CW)
    dst_p = jnp.concatenate(
        [dst, _N + 600 + (padi % 1024)]).reshape(_NW, _CHUNKS, _CW)

    ones16 = jnp.ones((_CW, 16), jnp.float32)
    z16 = jnp.zeros((_TROWS, 16), jnp.float32)
    z32 = jnp.zeros((_TROWS, 32), jnp.float32)

    deg_out = _deg_k(ones16, dst_p, z16)
    dis, xs1 = _dis_kernel(deg_out, xf)

    kt = lambda K, h: K[:, :, h, 0].T
    rs = lambda v: v.reshape(1, -1)
    agg1 = _agg16(xs1, src_p, dst_p, z16)
    y2a, y2b = _blk1(dis, xs1, agg1, W1, rs(b1), kt(K1, 0), kt(K1, 1),
                     kt(K1, 2), rs(kb1))
    ag2a = _agg32(y2a, src_p, dst_p, z32)
    ag2b = _agg32(y2b, src_p, dst_p, z32)
    y3a, y3b, y3c, y3d = _blk2(dis, y2a, y2b, ag2a, ag2b, W2, rs(b2),
                               kt(K2, 0), kt(K2, 1), kt(K2, 2), rs(kb2))
    ag3a = _agg32(y3a, src_p, dst_p, z32)
    ag3b = _agg32(y3b, src_p, dst_p, z32)
    ag3c = _agg32(y3c, src_p, dst_p, z32)
    ag3d = _agg32(y3d, src_p, dst_p, z32)
    cs = _blk3(dis, y3a, y3b, y3c, y3d, ag3a, ag3b, ag3c, ag3d, W3, rs(b3),
               kt(K3, 0), kt(K3, 1), kt(K3, 2), rs(kb3))[0]
    colmean = jnp.sum(cs, axis=0) * (1.0 / _N)
    return (colmean @ Wfc + bfc)[None, :]


# speedup vs baseline: 34.3042x; 1.2872x over previous
import functools

import jax
import jax.numpy as jnp
from jax import lax
from jax.experimental import pallas as pl
from jax.experimental.pallas import tpu as pltpu
from jax.experimental.pallas import tpu_sc as plsc

_N = 25000
_NTILES = 16
_TROWS = 1664
_NACC = _NTILES * _TROWS
_E = 400000
_CW = 128
_NW = 32
_NBUF = 4
_CHUNKS = 100
_EPAD = _NW * _CHUNKS * _CW
_R = 1000
_GRID = _N // _R

_SC_PARAMS = pltpu.CompilerParams(use_tc_tiling_on_sc=False)
_MESH = dict(core_axis_name="c", subcore_axis_name="s")


def _make_agg1(D, dtype=jnp.float32):
    @functools.partial(
        pl.kernel,
        out_type=jax.ShapeDtypeStruct((2, _NACC, D), dtype),
        mesh=plsc.VectorSubcoreMesh(**_MESH),
        compiler_params=_SC_PARAMS,
        scratch_types=[
            pltpu.VMEM((_CHUNKS, _CW), jnp.int32),
            pltpu.VMEM((_CHUNKS, _CW), jnp.int32),
            [pltpu.VMEM((_CW, D), dtype)] * _NBUF,
            [pltpu.SemaphoreType.DMA] * _NBUF,
            [pltpu.SemaphoreType.DMA] * _NBUF,
            pltpu.VMEM_SHARED((_NACC, D), dtype),
        ],
    )
    def agg(xs_hbm, src_hbm, dst_hbm, zero_hbm, out_hbm, srcb, dstb, gbufs,
            gsems, ssems, acc):
        cid = lax.axis_index("c")
        sid = lax.axis_index("s")
        wid = cid * _NTILES + sid
        lo = sid * _TROWS
        pltpu.sync_copy(src_hbm.at[wid], srcb)
        pltpu.sync_copy(dst_hbm.at[wid], dstb)
        pltpu.sync_copy(zero_hbm, acc.at[pl.ds(lo, _TROWS)])
        plsc.subcore_barrier()

        def gath(c, b):
            pltpu.async_copy(xs_hbm.at[srcb.at[c]], gbufs[b], gsems[b])

        def gath_wait(c, b):
            pltpu.make_async_copy(xs_hbm.at[srcb.at[c]], gbufs[b],
                                  gsems[b]).wait()

        def scat(c, b):
            pltpu.async_copy(gbufs[b], acc.at[dstb.at[c]], ssems[b],
                             add=True)

        def scat_wait(c, b):
            pltpu.make_async_copy(gbufs[b], acc.at[dstb.at[c]],
                                  ssems[b]).wait()

        for b in range(_NBUF):
            gath(b, b)

        def body(i, carry):
            c0 = i * _NBUF
            for b in range(_NBUF):
                gath_wait(c0 + b, b)
                scat(c0 + b, b)
            for b in range(_NBUF):
                scat_wait(c0 + b, b)
                gath(c0 + _NBUF + b, b)
            return carry

        lax.fori_loop(0, _CHUNKS // _NBUF - 1, body, 0)
        c0 = _CHUNKS - _NBUF
        for b in range(_NBUF):
            gath_wait(c0 + b, b)
            scat(c0 + b, b)
        for b in range(_NBUF):
            scat_wait(c0 + b, b)
        plsc.subcore_barrier()
        pltpu.sync_copy(acc.at[pl.ds(lo, _TROWS)],
                        out_hbm.at[cid, pl.ds(lo, _TROWS)])

    return agg


def _make_deg():
    @functools.partial(
        pl.kernel,
        out_type=jax.ShapeDtypeStruct((2, _NACC, 16), jnp.float32),
        mesh=plsc.VectorSubcoreMesh(**_MESH),
        compiler_params=_SC_PARAMS,
        scratch_types=[
            pltpu.VMEM((_CHUNKS, _CW), jnp.int32),
            pltpu.VMEM((_CW, 16), jnp.float32),
            pltpu.VMEM_SHARED((_NACC, 16), jnp.float32),
        ],
    )
    def deg(ones_hbm, dst_hbm, zero_hbm, out_hbm, dstb, obuf, acc):
        cid = lax.axis_index("c")
        sid = lax.axis_index("s")
        wid = cid * _NTILES + sid
        lo = sid * _TROWS
        pltpu.sync_copy(dst_hbm.at[wid], dstb)
        pltpu.sync_copy(ones_hbm, obuf)
        pltpu.sync_copy(zero_hbm, acc.at[pl.ds(lo, _TROWS)])
        plsc.subcore_barrier()

        def body(c, carry):
            pltpu.sync_copy(obuf, acc.at[dstb.at[c]], add=True)
            return carry

        lax.fori_loop(0, _CHUNKS, body, 0)
        plsc.subcore_barrier()
        pltpu.sync_copy(acc.at[pl.ds(lo, _TROWS)],
                        out_hbm.at[cid, pl.ds(lo, _TROWS)])

    return deg


def _dis_body(deg_ref, x_ref, dis_ref, xs_ref):
    d = deg_ref[0, :, 0:1] + deg_ref[1, :, 0:1] + 1.0
    dis = lax.rsqrt(d)
    dis_ref[...] = dis
    xs_ref[...] = dis * x_ref[...]


def _dis_kernel(deg_out, xf):
    return pl.pallas_call(
        _dis_body,
        grid=(_GRID,),
        in_specs=[
            pl.BlockSpec((2, _R, 16), lambda i: (0, i, 0)),
            pl.BlockSpec((_R, 16), lambda i: (i, 0)),
        ],
        out_specs=[
            pl.BlockSpec((_R, 1), lambda i: (i, 0)),
            pl.BlockSpec((_R, 16), lambda i: (i, 0)),
        ],
        out_shape=[
            jax.ShapeDtypeStruct((_N, 1), jnp.float32),
            jax.ShapeDtypeStruct((_N, 16), jnp.float32),
        ],
    )(deg_out, xf)


def _make_block(Din, Dout, n_in, n_out, in_dtype=jnp.float32,
                out_dtype=jnp.float32):
    P = Din // n_in

    def body(*refs):
        dis_ref = refs[0]
        xs_refs = refs[1:1 + n_in]
        ag_refs = refs[1 + n_in:1 + 2 * n_in]
        w_ref, b_ref, k0, k1, k2, kb_ref = refs[1 + 2 * n_in:7 + 2 * n_in]
        outs = refs[7 + 2 * n_in:]
        f32 = jnp.float32
        if n_in == 1:
            xs = xs_refs[0][...].astype(f32)
            a = ag_refs[0][0].astype(f32) + ag_refs[0][1].astype(f32)
        else:
            xs = jnp.concatenate(
                [r[...].astype(f32) for r in xs_refs], axis=-1)
            a = jnp.concatenate(
                [r[0].astype(f32) + r[1].astype(f32) for r in ag_refs],
                axis=-1)
        dis = dis_ref[...]
        m = dis * (a + xs)
        g = jnp.dot(m, w_ref[...], preferred_element_type=jnp.float32)
        g = g + b_ref[...]
        t = lax.broadcasted_iota(jnp.int32, (_R, 1), 0) % 10
        zrow = jnp.zeros((1, Dout), jnp.float32)
        gm1 = jnp.where(t == 0, 0.0,
                        jnp.concatenate([zrow, g[:-1]], axis=0))
        gp1 = jnp.where(t == 9, 0.0,
                        jnp.concatenate([g[1:], zrow], axis=0))
        y = (jnp.dot(g, k1[...], preferred_element_type=jnp.float32)
             + jnp.dot(gm1, k0[...], preferred_element_type=jnp.float32)
             + jnp.dot(gp1, k2[...], preferred_element_type=jnp.float32)
             + kb_ref[...])
        y = jnp.maximum(y, 0.0)
        if n_out == 0:
            ps = jnp.sum(y.reshape(_R // 8, 8, Dout), axis=0)
            i = pl.program_id(0)

            @pl.when(i == 0)
            def _():
                outs[0][...] = jnp.zeros_like(outs[0])

            outs[0][...] += ps
        else:
            ys = (dis * y).astype(out_dtype)
            q = Dout // n_out
            for j in range(n_out):
                outs[j][...] = ys[:, j * q:(j + 1) * q]

    full = lambda i: (0, 0)
    in_specs = [pl.BlockSpec((_R, 1), lambda i: (i, 0))]
    in_specs += [pl.BlockSpec((_R, P), lambda i: (i, 0))] * n_in
    in_specs += [pl.BlockSpec((2, _R, P), lambda i: (0, i, 0))] * n_in
    in_specs += [
        pl.BlockSpec((Din, Dout), full),
        pl.BlockSpec((1, Dout), full),
        pl.BlockSpec((Dout, Dout), full),
        pl.BlockSpec((Dout, Dout), full),
        pl.BlockSpec((Dout, Dout), full),
        pl.BlockSpec((1, Dout), full),
    ]
    if n_out == 0:
        out_specs = [pl.BlockSpec((8, Dout), full)]
        out_shape = [jax.ShapeDtypeStruct((8, Dout), jnp.float32)]
    else:
        q = Dout // n_out
        out_specs = [pl.BlockSpec((_R, q), lambda i: (i, 0))] * n_out
        out_shape = [jax.ShapeDtypeStruct((_N, q), out_dtype)] * n_out

    def run(*args):
        return pl.pallas_call(
            body,
            grid=(_GRID,),
            in_specs=in_specs,
            out_specs=out_specs,
            out_shape=out_shape,
        )(*args)

    return run


_deg_k = _make_deg()
_agg16 = _make_agg1(16)
_agg64b = _make_agg1(64, jnp.bfloat16)
_blk1 = _make_block(16, 64, 1, 1, out_dtype=jnp.bfloat16)
_blk2 = _make_block(64, 128, 1, 2, out_dtype=jnp.bfloat16)
_blk3 = _make_block(128, 256, 2, 0)


def kernel(x, edge_index, W1, b1, K1, kb1, W2, b2, K2, kb2, W3, b3, K3, kb3,
           Wfc, bfc):
    xf = x.reshape(_N, 16)
    src = edge_index[0].astype(jnp.int32)
    dst = edge_index[1].astype(jnp.int32)
    pad = _EPAD - _E
    padi = jnp.arange(pad, dtype=jnp.int32)
    src_p = jnp.concatenate([src, padi % _N]).reshape(_NW, _CHUNKS, _CW)
    dst_p = jnp.concatenate(
        [dst, _N + 600 + (padi % 1024)]).reshape(_NW, _CHUNKS, _CW)

    ones16 = jnp.ones((_CW, 16), jnp.float32)
    z16 = jnp.zeros((_TROWS, 16), jnp.float32)
    z64b = jnp.zeros((_TROWS, 64), jnp.bfloat16)

    deg_out = _deg_k(ones16, dst_p, z16)
    dis, xs1 = _dis_kernel(deg_out, xf)

    kt = lambda K, h: K[:, :, h, 0].T
    rs = lambda v: v.reshape(1, -1)
    agg1 = _agg16(xs1, src_p, dst_p, z16)
    y2 = _blk1(dis, xs1, agg1, W1, rs(b1), kt(K1, 0), kt(K1, 1),
               kt(K1, 2), rs(kb1))[0]
    ag2 = _agg64b(y2, src_p, dst_p, z64b)
    y3a, y3b = _blk2(dis, y2, ag2, W2, rs(b2),
                     kt(K2, 0), kt(K2, 1), kt(K2, 2), rs(kb2))
    ag3a = _agg64b(y3a, src_p, dst_p, z64b)
    ag3b = _agg64b(y3b, src_p, dst_p, z64b)
    cs = _blk3(dis, y3a, y3b, ag3a, ag3b, W3, rs(b3),
               kt(K3, 0), kt(K3, 1), kt(K3, 2), rs(kb3))[0]
    colmean = jnp.sum(cs, axis=0) * (1.0 / _N)
    return (colmean @ Wfc + bfc)[None, :]
